# baseline (speedup 1.0000x reference)
import functools

import numpy as np
import jax
import jax.numpy as jnp
from jax import lax
from jax.experimental import pallas as pl
from jax.experimental.pallas import tpu as pltpu
from jax.experimental.pallas import tpu_sc as plsc

N_TRIPLETS = 100000
NW = 32
CHUNK = 128
VREGS = CHUNK // 16
VALS_PAD = 19712

_trip_cache = {}


def _tf2x32(k1, k2, x1, x2):
    def rotl(x, d):
        return ((x << np.uint32(d)) | (x >> np.uint32(32 - d))).astype(np.uint32)

    ks0, ks1 = np.uint32(k1), np.uint32(k2)
    ks2 = np.uint32(ks0 ^ ks1 ^ np.uint32(0x1BD11BDA))
    rot_a = (13, 15, 26, 6)
    rot_b = (17, 29, 16, 24)
    x0 = (x1 + ks0).astype(np.uint32)
    x1 = (x2 + ks1).astype(np.uint32)
    sched = ((rot_a, ks1, ks2, 1), (rot_b, ks2, ks0, 2),
             (rot_a, ks0, ks1, 3), (rot_b, ks1, ks2, 4),
             (rot_a, ks2, ks0, 5))
    for rots, b0, b1, inc in sched:
        for r in rots:
            x0 = (x0 + x1).astype(np.uint32)
            x1 = rotl(x1, r) ^ x0
        x0 = (x0 + b0).astype(np.uint32)
        x1 = (x1 + b1 + np.uint32(inc)).astype(np.uint32)
    return x0, x1


def _tf_block(key, n):
    cnt = np.arange(n, dtype=np.uint32)
    return _tf2x32(key[0], key[1], np.zeros(n, np.uint32), cnt)


def _tf_bits(key, n):
    hi, lo = _tf_block(key, n)
    return hi ^ lo


def _tf_split(key, num):
    hi, lo = _tf_block(key, num)
    return np.stack([hi, lo], axis=1)


def _tf_randint(key, n, span):
    k_hi, k_lo = _tf_split(key, 2)
    higher = _tf_bits(k_hi, n)
    lower = _tf_bits(k_lo, n)
    m = np.uint32(span)
    mult = np.uint32((((65536 % span) * (65536 % span)) % (2**32)) % span)
    out = ((higher % m) * mult + (lower % m)) % m
    return out.astype(np.int32)


def _triplets(batch_size):
    if batch_size not in _trip_cache:
        n = min(N_TRIPLETS, batch_size * (batch_size - 1) * (batch_size - 2) // 6)
        key42 = np.array([0, 42], dtype=np.uint32)
        ka, kp, kn = _tf_split(key42, 3)
        a = _tf_randint(ka, n, batch_size)
        p = _tf_randint(kp, n, batch_size)
        q = _tf_randint(kn, n, batch_size)
        per_w = -(-n // (NW * 4 * CHUNK)) * 4 * CHUNK
        n_pad = per_w * NW
        ap = np.zeros(n_pad, np.int32)
        pp = np.zeros(n_pad, np.int32)
        qp = np.zeros(n_pad, np.int32)
        ap[:n] = a
        pp[:n] = p
        qp[:n] = q
        _trip_cache[batch_size] = (ap, pp, qp, per_w)
    return _trip_cache[batch_size]


def _nsqrt(x):
    xi = lax.bitcast_convert_type(x, jnp.int32)
    r = lax.bitcast_convert_type(0x5F3759DF - (xi >> 1), jnp.float32)
    xh = 0.5 * x
    r = r * (1.5 - xh * r * r)
    r = r * (1.5 - xh * r * r)
    r = r * (1.5 - xh * r * r)
    return x * r


@functools.lru_cache(maxsize=4)
def _build(batch_size, dim, per_w, n_vals):
    n_chunks = per_w // CHUNK
    mesh = plsc.VectorSubcoreMesh(core_axis_name="c", subcore_axis_name="s")

    @functools.partial(
        pl.kernel,
        mesh=mesh,
        compiler_params=pltpu.CompilerParams(
            needs_layout_passes=False, use_tc_tiling_on_sc=False),
        out_type=[
            jax.ShapeDtypeStruct((NW, 16), jnp.float32),
            jax.ShapeDtypeStruct((NW, 16), jnp.float32),
        ],
        scratch_types=[
            pltpu.VMEM((batch_size,), jnp.int32),
            pltpu.VMEM((n_vals,), jnp.int32),
            pltpu.VMEM((per_w,), jnp.int32),
            pltpu.VMEM((per_w,), jnp.int32),
            pltpu.VMEM((per_w,), jnp.int32),
            pltpu.VMEM((2 * 512, dim), jnp.float32),
            pltpu.VMEM((2 * 512, dim), jnp.float32),
            pltpu.VMEM((2 * 512, dim), jnp.float32),
            pltpu.VMEM((16,), jnp.float32),
            pltpu.VMEM((16,), jnp.float32),
            pltpu.VMEM_SHARED((batch_size, dim), jnp.float32),
            pltpu.SemaphoreType.DMA,
            pltpu.SemaphoreType.DMA,
        ],
    )
    def sc_kernel(z_hbm, ind_hbm, vals_hbm, ta_hbm, tp_hbm, tn_hbm,
                  out_s, out_c,
                  ind_t, vals_t, ai, pi, ni, za, zp, zn, sv, cv, z_sp,
                  sem0, sem1):
        sid = lax.axis_index("s")
        wid = sid * 2 + lax.axis_index("c")
        base = wid * per_w
        piece = 512
        n_pieces = per_w // piece
        n_fire = piece // CHUNK
        sems = (sem0, sem1)
        rows_per = batch_size // 16
        pltpu.sync_copy(ta_hbm.at[pl.ds(base, per_w)], ai)
        pltpu.sync_copy(tp_hbm.at[pl.ds(base, per_w)], pi)
        pltpu.sync_copy(tn_hbm.at[pl.ds(base, per_w)], ni)
        pltpu.sync_copy(z_hbm.at[pl.ds(sid * rows_per, rows_per)],
                        z_sp.at[pl.ds(sid * rows_per, rows_per)])
        plsc.subcore_barrier()
        lane = lax.broadcasted_iota(jnp.int32, (16,), 0)

        def fire(h):
            slot = h % 2
            hnd = []
            for c in range(n_fire):
                off = h * piece + c * CHUNK
                dst = pl.ds(slot * piece + c * CHUNK, CHUNK)
                hnd.append(pltpu.async_copy(
                    z_sp.at[ai.at[pl.ds(off, CHUNK)]], za.at[dst], sems[slot]))
                hnd.append(pltpu.async_copy(
                    z_sp.at[pi.at[pl.ds(off, CHUNK)]], zp.at[dst], sems[slot]))
                hnd.append(pltpu.async_copy(
                    z_sp.at[ni.at[pl.ds(off, CHUNK)]], zn.at[dst], sems[slot]))
            return hnd

        handles = [None, None]
        handles[0] = fire(0)
        pltpu.sync_copy(ind_hbm, ind_t)
        pltpu.sync_copy(vals_hbm, vals_t)

        acc_s = jnp.zeros((16,), jnp.float32)
        acc_c = jnp.zeros((16,), jnp.float32)
        for h in range(n_pieces):
            slot = h % 2
            if h + 1 < n_pieces:
                handles[(h + 1) % 2] = fire(h + 1)
            for hd in handles[slot]:
                hd.wait()

            def one_vreg(t0, rows):
                av = ai[pl.ds(t0, 16)]
                pv = pi[pl.ds(t0, 16)]
                nv = ni[pl.ds(t0, 16)]
                oa = plsc.load_gather(ind_t, [av])
                op = plsc.load_gather(ind_t, [pv])
                on = plsc.load_gather(ind_t, [nv])
                dp = jnp.minimum(jnp.abs(oa - op), n_vals - 1)
                dn = jnp.minimum(jnp.abs(oa - on), n_vals - 1)
                vp = plsc.load_gather(vals_t, [dp])
                vn = plsc.load_gather(vals_t, [dn])
                valid = (vp > vn) & (av != pv) & (av != nv)
                sqp = jnp.zeros((16,), jnp.float32)
                sqn = jnp.zeros((16,), jnp.float32)
                for kk in range(dim):
                    col = jnp.full((16,), kk, jnp.int32)
                    xa = plsc.load_gather(za, [rows, col])
                    xp = plsc.load_gather(zp, [rows, col])
                    xn = plsc.load_gather(zn, [rows, col])
                    dpos = xa - xp
                    dneg = xa - xn
                    sqp = sqp + dpos * dpos
                    sqn = sqn + dneg * dneg
                marg = 0.1 + 0.05 * jnp.abs(vp - vn).astype(jnp.float32)
                per = jnp.maximum(_nsqrt(sqp) - _nsqrt(sqn) + marg, 0.0)
                vf = jnp.where(valid, 1.0, 0.0).astype(jnp.float32)
                return per * vf, vf

            def vbody(v, carry, _hb=h * piece, _rb=slot * piece):
                a_s, a_c = carry
                s0, c0 = one_vreg(_hb + v * 32, _rb + v * 32 + lane)
                s1, c1 = one_vreg(_hb + v * 32 + 16, _rb + v * 32 + 16 + lane)
                return a_s + (s0 + s1), a_c + (c0 + c1)

            acc_s, acc_c = lax.fori_loop(0, piece // 32, vbody, (acc_s, acc_c))
        sv[...] = acc_s
        cv[...] = acc_c
        pltpu.sync_copy(sv, out_s.at[wid])
        pltpu.sync_copy(cv, out_c.at[wid])

    return sc_kernel


def kernel(z, indices, valuations):
    batch_size, dim = z.shape
    ta, tp, tn, per_w = _triplets(batch_size)
    vals_p = jnp.pad(valuations, (0, VALS_PAD - valuations.shape[0]))
    fn = _build(batch_size, dim, per_w, VALS_PAD)
    sums, cnts = fn(z, indices, vals_p,
                    jnp.asarray(ta), jnp.asarray(tp), jnp.asarray(tn))
    s = jnp.sum(sums)
    c = jnp.sum(cnts)
    return jnp.where(c > 0, s / jnp.maximum(c, 1.0), 0.0)


# speedup vs baseline: 1.1156x; 1.1156x over previous
import functools

import numpy as np
import jax
import jax.numpy as jnp
from jax import lax
from jax.experimental import pallas as pl
from jax.experimental.pallas import tpu as pltpu
from jax.experimental.pallas import tpu_sc as plsc

N_TRIPLETS = 100000
NW = 32
CHUNK = 128
VREGS = CHUNK // 16
VALS_PAD = 19712

_trip_cache = {}


def _tf2x32(k1, k2, x1, x2):
    def rotl(x, d):
        return ((x << np.uint32(d)) | (x >> np.uint32(32 - d))).astype(np.uint32)

    ks0, ks1 = np.uint32(k1), np.uint32(k2)
    ks2 = np.uint32(ks0 ^ ks1 ^ np.uint32(0x1BD11BDA))
    rot_a = (13, 15, 26, 6)
    rot_b = (17, 29, 16, 24)
    x0 = (x1 + ks0).astype(np.uint32)
    x1 = (x2 + ks1).astype(np.uint32)
    sched = ((rot_a, ks1, ks2, 1), (rot_b, ks2, ks0, 2),
             (rot_a, ks0, ks1, 3), (rot_b, ks1, ks2, 4),
             (rot_a, ks2, ks0, 5))
    for rots, b0, b1, inc in sched:
        for r in rots:
            x0 = (x0 + x1).astype(np.uint32)
            x1 = rotl(x1, r) ^ x0
        x0 = (x0 + b0).astype(np.uint32)
        x1 = (x1 + b1 + np.uint32(inc)).astype(np.uint32)
    return x0, x1


def _tf_block(key, n):
    cnt = np.arange(n, dtype=np.uint32)
    return _tf2x32(key[0], key[1], np.zeros(n, np.uint32), cnt)


def _tf_bits(key, n):
    hi, lo = _tf_block(key, n)
    return hi ^ lo


def _tf_split(key, num):
    hi, lo = _tf_block(key, num)
    return np.stack([hi, lo], axis=1)


def _tf_randint(key, n, span):
    k_hi, k_lo = _tf_split(key, 2)
    higher = _tf_bits(k_hi, n)
    lower = _tf_bits(k_lo, n)
    m = np.uint32(span)
    mult = np.uint32((((65536 % span) * (65536 % span)) % (2**32)) % span)
    out = ((higher % m) * mult + (lower % m)) % m
    return out.astype(np.int32)


def _triplets(batch_size):
    if batch_size not in _trip_cache:
        n = min(N_TRIPLETS, batch_size * (batch_size - 1) * (batch_size - 2) // 6)
        key42 = np.array([0, 42], dtype=np.uint32)
        ka, kp, kn = _tf_split(key42, 3)
        a = _tf_randint(ka, n, batch_size)
        p = _tf_randint(kp, n, batch_size)
        q = _tf_randint(kn, n, batch_size)
        per_w = -(-n // (NW * 2 * CHUNK)) * 2 * CHUNK
        n_pad = per_w * NW
        ap = np.zeros(n_pad, np.int32)
        pp = np.zeros(n_pad, np.int32)
        qp = np.zeros(n_pad, np.int32)
        ap[:n] = a
        pp[:n] = p
        qp[:n] = q
        _trip_cache[batch_size] = (ap, pp, qp, per_w)
    return _trip_cache[batch_size]


def _nsqrt(x):
    xi = lax.bitcast_convert_type(x, jnp.int32)
    r = lax.bitcast_convert_type(0x5F3759DF - (xi >> 1), jnp.float32)
    xh = 0.5 * x
    r = r * (1.5 - xh * r * r)
    r = r * (1.5 - xh * r * r)
    r = r * (1.5 - xh * r * r)
    return x * r


@functools.lru_cache(maxsize=4)
def _build(batch_size, dim, per_w, n_vals):
    n_chunks = per_w // CHUNK
    mesh = plsc.VectorSubcoreMesh(core_axis_name="c", subcore_axis_name="s")

    @functools.partial(
        pl.kernel,
        mesh=mesh,
        compiler_params=pltpu.CompilerParams(
            needs_layout_passes=False, use_tc_tiling_on_sc=False),
        out_type=[
            jax.ShapeDtypeStruct((NW, 16), jnp.float32),
            jax.ShapeDtypeStruct((NW, 16), jnp.float32),
        ],
        scratch_types=[
            pltpu.VMEM((batch_size,), jnp.int32),
            pltpu.VMEM((n_vals,), jnp.int32),
            pltpu.VMEM((per_w,), jnp.int32),
            pltpu.VMEM((per_w,), jnp.int32),
            pltpu.VMEM((per_w,), jnp.int32),
            pltpu.VMEM((2 * 256, dim), jnp.float32),
            pltpu.VMEM((2 * 256, dim), jnp.float32),
            pltpu.VMEM((2 * 256, dim), jnp.float32),
            pltpu.VMEM((16,), jnp.float32),
            pltpu.VMEM((16,), jnp.float32),
            pltpu.VMEM_SHARED((batch_size, dim), jnp.float32),
            pltpu.VMEM_SHARED((batch_size,), jnp.int32),
            pltpu.VMEM_SHARED((n_vals,), jnp.int32),
            pltpu.SemaphoreType.DMA,
            pltpu.SemaphoreType.DMA,
        ],
    )
    def sc_kernel(z_hbm, ind_hbm, vals_hbm, ta_hbm, tp_hbm, tn_hbm,
                  out_s, out_c,
                  ind_t, vals_t, ai, pi, ni, za, zp, zn, sv, cv, z_sp,
                  ind_sp, vals_sp, sem0, sem1):
        sid = lax.axis_index("s")
        wid = sid * 2 + lax.axis_index("c")
        base = wid * per_w
        piece = 256
        n_pieces = per_w // piece
        n_fire = piece // CHUNK
        sems = (sem0, sem1)
        rows_per = batch_size // 16
        pltpu.sync_copy(ta_hbm.at[pl.ds(base, per_w)], ai)
        pltpu.sync_copy(tp_hbm.at[pl.ds(base, per_w)], pi)
        pltpu.sync_copy(tn_hbm.at[pl.ds(base, per_w)], ni)
        pltpu.sync_copy(z_hbm.at[pl.ds(sid * rows_per, rows_per)],
                        z_sp.at[pl.ds(sid * rows_per, rows_per)])
        ind_per = batch_size // 16
        pltpu.sync_copy(ind_hbm.at[pl.ds(sid * ind_per, ind_per)],
                        ind_sp.at[pl.ds(sid * ind_per, ind_per)])
        vals_per = n_vals // 16
        pltpu.sync_copy(vals_hbm.at[pl.ds(sid * vals_per, vals_per)],
                        vals_sp.at[pl.ds(sid * vals_per, vals_per)])
        plsc.subcore_barrier()
        lane = lax.broadcasted_iota(jnp.int32, (16,), 0)

        def fire(h):
            slot = h % 2
            hnd = []
            for c in range(n_fire):
                off = h * piece + c * CHUNK
                dst = pl.ds(slot * piece + c * CHUNK, CHUNK)
                hnd.append(pltpu.async_copy(
                    z_sp.at[ai.at[pl.ds(off, CHUNK)]], za.at[dst], sems[slot]))
                hnd.append(pltpu.async_copy(
                    z_sp.at[pi.at[pl.ds(off, CHUNK)]], zp.at[dst], sems[slot]))
                hnd.append(pltpu.async_copy(
                    z_sp.at[ni.at[pl.ds(off, CHUNK)]], zn.at[dst], sems[slot]))
            return hnd

        handles = [None, None]
        handles[0] = fire(0)
        pltpu.sync_copy(ind_sp, ind_t)
        pltpu.sync_copy(vals_sp, vals_t)

        acc_s = jnp.zeros((16,), jnp.float32)
        acc_c = jnp.zeros((16,), jnp.float32)
        for h in range(n_pieces):
            slot = h % 2
            if h + 1 < n_pieces:
                handles[(h + 1) % 2] = fire(h + 1)
            for hd in handles[slot]:
                hd.wait()

            def one_vreg(t0, rows):
                av = ai[pl.ds(t0, 16)]
                pv = pi[pl.ds(t0, 16)]
                nv = ni[pl.ds(t0, 16)]
                oa = plsc.load_gather(ind_t, [av])
                op = plsc.load_gather(ind_t, [pv])
                on = plsc.load_gather(ind_t, [nv])
                dp = jnp.minimum(jnp.abs(oa - op), n_vals - 1)
                dn = jnp.minimum(jnp.abs(oa - on), n_vals - 1)
                vp = plsc.load_gather(vals_t, [dp])
                vn = plsc.load_gather(vals_t, [dn])
                valid = (vp > vn) & (av != pv) & (av != nv)
                sqp = jnp.zeros((16,), jnp.float32)
                sqn = jnp.zeros((16,), jnp.float32)
                for kk in range(dim):
                    col = jnp.full((16,), kk, jnp.int32)
                    xa = plsc.load_gather(za, [rows, col])
                    xp = plsc.load_gather(zp, [rows, col])
                    xn = plsc.load_gather(zn, [rows, col])
                    dpos = xa - xp
                    dneg = xa - xn
                    sqp = sqp + dpos * dpos
                    sqn = sqn + dneg * dneg
                marg = 0.1 + 0.05 * jnp.abs(vp - vn).astype(jnp.float32)
                per = jnp.maximum(_nsqrt(sqp) - _nsqrt(sqn) + marg, 0.0)
                vf = jnp.where(valid, 1.0, 0.0).astype(jnp.float32)
                return per * vf, vf

            def vbody(v, carry, _hb=h * piece, _rb=slot * piece):
                a_s, a_c = carry
                s0, c0 = one_vreg(_hb + v * 16, _rb + v * 16 + lane)
                return a_s + s0, a_c + c0

            acc_s, acc_c = lax.fori_loop(0, piece // 16, vbody, (acc_s, acc_c))
        sv[...] = acc_s
        cv[...] = acc_c
        pltpu.sync_copy(sv, out_s.at[wid])
        pltpu.sync_copy(cv, out_c.at[wid])

    return sc_kernel


def kernel(z, indices, valuations):
    batch_size, dim = z.shape
    ta, tp, tn, per_w = _triplets(batch_size)
    vals_p = jnp.pad(valuations, (0, VALS_PAD - valuations.shape[0]))
    fn = _build(batch_size, dim, per_w, VALS_PAD)
    sums, cnts = fn(z, indices, vals_p,
                    jnp.asarray(ta), jnp.asarray(tp), jnp.asarray(tn))
    s = jnp.sum(sums)
    c = jnp.sum(cnts)
    return jnp.where(c > 0, s / jnp.maximum(c, 1.0), 0.0)


# speedup vs baseline: 1.1161x; 1.0005x over previous
import functools

import numpy as np
import jax
import jax.numpy as jnp
from jax import lax
from jax.experimental import pallas as pl
from jax.experimental.pallas import tpu as pltpu
from jax.experimental.pallas import tpu_sc as plsc

N_TRIPLETS = 100000
NW = 32
CHUNK = 128
VREGS = CHUNK // 16
VALS_PAD = 19712

_trip_cache = {}


def _tf2x32(k1, k2, x1, x2):
    def rotl(x, d):
        return ((x << np.uint32(d)) | (x >> np.uint32(32 - d))).astype(np.uint32)

    ks0, ks1 = np.uint32(k1), np.uint32(k2)
    ks2 = np.uint32(ks0 ^ ks1 ^ np.uint32(0x1BD11BDA))
    rot_a = (13, 15, 26, 6)
    rot_b = (17, 29, 16, 24)
    x0 = (x1 + ks0).astype(np.uint32)
    x1 = (x2 + ks1).astype(np.uint32)
    sched = ((rot_a, ks1, ks2, 1), (rot_b, ks2, ks0, 2),
             (rot_a, ks0, ks1, 3), (rot_b, ks1, ks2, 4),
             (rot_a, ks2, ks0, 5))
    for rots, b0, b1, inc in sched:
        for r in rots:
            x0 = (x0 + x1).astype(np.uint32)
            x1 = rotl(x1, r) ^ x0
        x0 = (x0 + b0).astype(np.uint32)
        x1 = (x1 + b1 + np.uint32(inc)).astype(np.uint32)
    return x0, x1


def _tf_block(key, n):
    cnt = np.arange(n, dtype=np.uint32)
    return _tf2x32(key[0], key[1], np.zeros(n, np.uint32), cnt)


def _tf_bits(key, n):
    hi, lo = _tf_block(key, n)
    return hi ^ lo


def _tf_split(key, num):
    hi, lo = _tf_block(key, num)
    return np.stack([hi, lo], axis=1)


def _tf_randint(key, n, span):
    k_hi, k_lo = _tf_split(key, 2)
    higher = _tf_bits(k_hi, n)
    lower = _tf_bits(k_lo, n)
    m = np.uint32(span)
    mult = np.uint32((((65536 % span) * (65536 % span)) % (2**32)) % span)
    out = ((higher % m) * mult + (lower % m)) % m
    return out.astype(np.int32)


def _triplets(batch_size):
    if batch_size not in _trip_cache:
        n = min(N_TRIPLETS, batch_size * (batch_size - 1) * (batch_size - 2) // 6)
        key42 = np.array([0, 42], dtype=np.uint32)
        ka, kp, kn = _tf_split(key42, 3)
        a = _tf_randint(ka, n, batch_size)
        p = _tf_randint(kp, n, batch_size)
        q = _tf_randint(kn, n, batch_size)
        per_w = -(-n // (NW * 2 * CHUNK)) * 2 * CHUNK
        n_pad = per_w * NW
        ap = np.zeros(n_pad, np.int32)
        pp = np.zeros(n_pad, np.int32)
        qp = np.zeros(n_pad, np.int32)
        ap[:n] = a
        pp[:n] = p
        qp[:n] = q
        _trip_cache[batch_size] = (ap, pp, qp, per_w)
    return _trip_cache[batch_size]


def _nsqrt(x):
    xi = lax.bitcast_convert_type(x, jnp.int32)
    r = lax.bitcast_convert_type(0x5F3759DF - (xi >> 1), jnp.float32)
    xh = 0.5 * x
    r = r * (1.5 - xh * r * r)
    r = r * (1.5 - xh * r * r)
    r = r * (1.5 - xh * r * r)
    return x * r


@functools.lru_cache(maxsize=4)
def _build(batch_size, dim, per_w, n_vals):
    n_chunks = per_w // CHUNK
    mesh = plsc.VectorSubcoreMesh(core_axis_name="c", subcore_axis_name="s")

    @functools.partial(
        pl.kernel,
        mesh=mesh,
        compiler_params=pltpu.CompilerParams(
            needs_layout_passes=False, use_tc_tiling_on_sc=False),
        out_type=[
            jax.ShapeDtypeStruct((NW, 16), jnp.float32),
            jax.ShapeDtypeStruct((NW, 16), jnp.float32),
        ],
        scratch_types=[
            pltpu.VMEM((batch_size,), jnp.int32),
            pltpu.VMEM((n_vals,), jnp.int32),
            pltpu.VMEM((per_w,), jnp.int32),
            pltpu.VMEM((per_w,), jnp.int32),
            pltpu.VMEM((per_w,), jnp.int32),
            pltpu.VMEM((2 * 256, dim), jnp.float32),
            pltpu.VMEM((2 * 256, dim), jnp.float32),
            pltpu.VMEM((2 * 256, dim), jnp.float32),
            pltpu.VMEM((16,), jnp.float32),
            pltpu.VMEM((16,), jnp.float32),
            pltpu.VMEM_SHARED((batch_size, dim), jnp.float32),
            pltpu.VMEM_SHARED((batch_size,), jnp.int32),
            pltpu.VMEM_SHARED((n_vals,), jnp.int32),
            pltpu.SemaphoreType.DMA,
            pltpu.SemaphoreType.DMA,
        ],
    )
    def sc_kernel(z_hbm, ind_hbm, vals_hbm, ta_hbm, tp_hbm, tn_hbm,
                  out_s, out_c,
                  ind_t, vals_t, ai, pi, ni, za, zp, zn, sv, cv, z_sp,
                  ind_sp, vals_sp, sem0, sem1):
        sid = lax.axis_index("s")
        wid = sid * 2 + lax.axis_index("c")
        base = wid * per_w
        piece = 256
        n_pieces = per_w // piece
        n_fire = piece // CHUNK
        sems = (sem0, sem1)
        rows_per = batch_size // 16
        pltpu.sync_copy(ta_hbm.at[pl.ds(base, per_w)], ai)
        pltpu.sync_copy(tp_hbm.at[pl.ds(base, per_w)], pi)
        pltpu.sync_copy(tn_hbm.at[pl.ds(base, per_w)], ni)
        pltpu.sync_copy(z_hbm.at[pl.ds(sid * rows_per, rows_per)],
                        z_sp.at[pl.ds(sid * rows_per, rows_per)])
        ind_per = batch_size // 16
        pltpu.sync_copy(ind_hbm.at[pl.ds(sid * ind_per, ind_per)],
                        ind_sp.at[pl.ds(sid * ind_per, ind_per)])
        vals_per = n_vals // 16
        pltpu.sync_copy(vals_hbm.at[pl.ds(sid * vals_per, vals_per)],
                        vals_sp.at[pl.ds(sid * vals_per, vals_per)])
        plsc.subcore_barrier()
        lane = lax.broadcasted_iota(jnp.int32, (16,), 0)

        def fire(h):
            slot = h % 2
            hnd = []
            for c in range(n_fire):
                off = h * piece + c * CHUNK
                dst = pl.ds(slot * piece + c * CHUNK, CHUNK)
                hnd.append(pltpu.async_copy(
                    z_sp.at[ai.at[pl.ds(off, CHUNK)]], za.at[dst], sems[slot]))
                hnd.append(pltpu.async_copy(
                    z_sp.at[pi.at[pl.ds(off, CHUNK)]], zp.at[dst], sems[slot]))
                hnd.append(pltpu.async_copy(
                    z_sp.at[ni.at[pl.ds(off, CHUNK)]], zn.at[dst], sems[slot]))
            return hnd

        handles = [None, None]
        handles[0] = fire(0)
        pltpu.sync_copy(ind_sp, ind_t)
        pltpu.sync_copy(vals_sp, vals_t)

        acc_s = jnp.zeros((16,), jnp.float32)
        acc_c = jnp.zeros((16,), jnp.float32)
        for h in range(n_pieces):
            slot = h % 2
            if h + 1 < n_pieces:
                handles[(h + 1) % 2] = fire(h + 1)
            for hd in handles[slot]:
                hd.wait()

            def one_vreg(t0, rows):
                av = ai[pl.ds(t0, 16)]
                pv = pi[pl.ds(t0, 16)]
                nv = ni[pl.ds(t0, 16)]
                oa = plsc.load_gather(ind_t, [av])
                op = plsc.load_gather(ind_t, [pv])
                on = plsc.load_gather(ind_t, [nv])
                dp = jnp.minimum(jnp.abs(oa - op), n_vals - 1)
                dn = jnp.minimum(jnp.abs(oa - on), n_vals - 1)
                vp = plsc.load_gather(vals_t, [dp])
                vn = plsc.load_gather(vals_t, [dn])
                valid = (vp > vn) & (av != pv) & (av != nv)
                sqp = jnp.zeros((16,), jnp.float32)
                sqn = jnp.zeros((16,), jnp.float32)
                for kk in range(dim):
                    col = jnp.full((16,), kk, jnp.int32)
                    xa = plsc.load_gather(za, [rows, col])
                    xp = plsc.load_gather(zp, [rows, col])
                    xn = plsc.load_gather(zn, [rows, col])
                    dpos = xa - xp
                    dneg = xa - xn
                    sqp = sqp + dpos * dpos
                    sqn = sqn + dneg * dneg
                marg = 0.1 + 0.05 * jnp.abs(vp - vn).astype(jnp.float32)
                per = jnp.maximum(_nsqrt(sqp) - _nsqrt(sqn) + marg, 0.0)
                vf = jnp.where(valid, 1.0, 0.0).astype(jnp.float32)
                return per * vf, vf

            @plsc.parallel_loop(0, piece // 16, carry=(acc_s, acc_c))
            def _ploop(v, carry, _hb=h * piece, _rb=slot * piece):
                a_s, a_c = carry
                s0, c0 = one_vreg(_hb + v * 16, _rb + v * 16 + lane)
                return a_s + s0, a_c + c0

            acc_s, acc_c = _ploop
        sv[...] = acc_s
        cv[...] = acc_c
        pltpu.sync_copy(sv, out_s.at[wid])
        pltpu.sync_copy(cv, out_c.at[wid])

    return sc_kernel


def kernel(z, indices, valuations):
    batch_size, dim = z.shape
    ta, tp, tn, per_w = _triplets(batch_size)
    vals_p = jnp.pad(valuations, (0, VALS_PAD - valuations.shape[0]))
    fn = _build(batch_size, dim, per_w, VALS_PAD)
    sums, cnts = fn(z, indices, vals_p,
                    jnp.asarray(ta), jnp.asarray(tp), jnp.asarray(tn))
    s = jnp.sum(sums)
    c = jnp.sum(cnts)
    return jnp.where(c > 0, s / jnp.maximum(c, 1.0), 0.0)
